# Initial kernel scaffold; baseline (speedup 1.0000x reference)
import jax
import jax.numpy as jnp
from jax.experimental import pallas as pl


def kernel(edge_attr, W0, W1, W2):
    raise NotImplementedError("write your pallas kernel here")


# speedup vs baseline: 1.3449x; 1.3449x over previous
import functools

import jax
import jax.numpy as jnp
from jax import lax
from jax.experimental import pallas as pl
from jax.experimental.pallas import tpu as pltpu
from jax.experimental.pallas import tpu_sc as plsc

D0, D1, D2 = 22, 6, 2
ED = 32
OD = 3 * ED
NT = D0 * D1 * D2
NC, NS = 2, 16
NW = NC * NS
CH = 128
L = 16


def _sc_lookup(edge_attr, table):
    E = edge_attr.shape[0]
    assert E % NW == 0
    epw = E // NW
    nfull = epw // CH
    tail = epw - nfull * CH
    assert tail % L == 0

    mesh = plsc.VectorSubcoreMesh(core_axis_name="c", subcore_axis_name="s")

    @functools.partial(
        pl.kernel,
        out_type=jax.ShapeDtypeStruct((E, OD), jnp.float32),
        mesh=mesh,
        scratch_types=[
            pltpu.VMEM((CH * 3,), jnp.int32),
            pltpu.VMEM((CH,), jnp.int32),
            pltpu.VMEM((CH, OD), jnp.float32),
            pltpu.SemaphoreType.DMA,
        ],
        compiler_params=pltpu.CompilerParams(
            needs_layout_passes=False, use_tc_tiling_on_sc=False),
    )
    def k(attr_hbm, tab_hbm, out_hbm, attr_v, idx_v, rows_v, sem):
        wid = lax.axis_index("s") * NC + lax.axis_index("c")
        base = wid * epw
        zeros = jnp.zeros((L,), jnp.int32)
        iota3 = lax.iota(jnp.int32, L) * 3

        def compute_idx(i):
            b = jnp.full((L,), i * L * 3, jnp.int32) + iota3
            e0 = plsc.load_gather(attr_v, [b])
            e1 = plsc.load_gather(attr_v, [b + 1])
            e2 = plsc.load_gather(attr_v, [b + 2])
            e0 = jnp.maximum(e0, 0)
            e1 = jnp.clip(e1, 0, D1 - 1)
            e2 = jnp.clip(e2, 0, D2 - 1)
            idx_v[pl.ds(i * L, L)] = e0 * (D1 * D2) + e1 * D2 + e2

        def do_chunk(cbase, n):
            pltpu.sync_copy(attr_hbm.at[pl.ds(cbase * 3, n * 3)],
                            attr_v.at[pl.ds(0, n * 3)])
            for i in range(n // L):
                compute_idx(i)
            for i in range(n // L, CH // L):
                idx_v[pl.ds(i * L, L)] = zeros
            pltpu.async_copy(tab_hbm.at[idx_v], rows_v, sem).wait()
            pltpu.sync_copy(rows_v.at[pl.ds(0, n)],
                            out_hbm.at[pl.ds(cbase, n)])

        def body(j, carry):
            do_chunk(base + j * CH, CH)
            return carry

        lax.fori_loop(0, nfull, body, 0)
        if tail:
            do_chunk(base + nfull * CH, tail)

    return k(edge_attr.reshape(-1), table)


def kernel(edge_attr, W0, W1, W2):
    r = jnp.arange(NT, dtype=jnp.int32)
    table = jnp.concatenate(
        [W0[r // (D1 * D2)], W1[(r // D2) % D1], W2[r % D2]], axis=-1)
    return _sc_lookup(edge_attr, table)
